# baseline (speedup 1.0000x reference)
import functools
import math

import jax
import jax.numpy as jnp
from jax.experimental import pallas as pl
from jax.experimental.pallas import tpu as pltpu

_NUM_UNITS = 1024
_SCALE = float(_NUM_UNITS) ** 0.5
_NEG2LOG1E4_OVER_D = -2.0 * math.log(10000.0) / _NUM_UNITS
_HALF_PI = math.pi / 2.0
_SUB = 64
_STRIPE = 128


def _pe_body(out_ref, table_ref, s64_ref, c64_ref, sem_ref, *, seq: int, batch: int):
    ch1 = jax.lax.broadcasted_iota(jnp.int32, (1, _NUM_UNITS), 1)
    w1 = jnp.exp(ch1.astype(jnp.float32) * _NEG2LOG1E4_OVER_D)
    off = (ch1 % 2).astype(jnp.float32) * _HALF_PI

    r8 = jax.lax.broadcasted_iota(jnp.int32, (8, _NUM_UNITS), 0)
    ch8 = jax.lax.broadcasted_iota(jnp.int32, (8, _NUM_UNITS), 1)
    rw8 = r8.astype(jnp.float32) * jnp.exp(
        ch8.astype(jnp.float32) * _NEG2LOG1E4_OVER_D
    )
    s8 = jnp.sin(rw8)
    c8 = jnp.sin(rw8 + _HALF_PI)
    s64_ref[0:8, :] = s8
    c64_ref[0:8, :] = c8
    for q in range(1, _SUB // 8):
        ph = float(8 * q) * w1
        sq = jnp.sin(ph)
        cq = jnp.sin(ph + _HALF_PI)
        s64_ref[8 * q : 8 * q + 8, :] = sq * c8 + cq * s8
        c64_ref[8 * q : 8 * q + 8, :] = cq * c8 - sq * s8

    s64 = s64_ref[...]
    c64 = c64_ref[...]

    n_stripes = seq // _STRIPE

    def _copies(s):
        return [
            pltpu.make_async_copy(
                table_ref.at[pl.ds(s * _STRIPE, _STRIPE), :],
                out_ref.at[n, pl.ds(s * _STRIPE, _STRIPE), :],
                sem_ref.at[s % 5, n],
            )
            for n in range(batch)
        ]

    for s in range(n_stripes):
        for a in range(_STRIPE // _SUB):
            p0 = s * _STRIPE + a * _SUB
            phase = float(p0) * w1 + off
            sb = jnp.sin(phase) * _SCALE
            cb = jnp.sin(phase + _HALF_PI) * _SCALE
            row = pl.ds(p0, _SUB)
            table_ref[row, :] = sb * c64 + cb * s64
        if s == 0:
            table_ref[0:1, :] = jnp.zeros((1, _NUM_UNITS), jnp.float32)
        for c in _copies(s):
            c.start()
        if s >= 4:
            for c in _copies(s - 4):
                c.wait()
    for s in range(n_stripes - 4, n_stripes):
        for c in _copies(s):
            c.wait()


def kernel(inputs):
    n, t, d = inputs.shape
    body = functools.partial(_pe_body, seq=t, batch=n)
    return pl.pallas_call(
        body,
        out_shape=jax.ShapeDtypeStruct((n, t, d), jnp.float32),
        out_specs=pl.BlockSpec(memory_space=pl.ANY),
        scratch_shapes=[
            pltpu.VMEM((t, d), jnp.float32),
            pltpu.VMEM((_SUB, d), jnp.float32),
            pltpu.VMEM((_SUB, d), jnp.float32),
            pltpu.SemaphoreType.DMA((5, n)),
        ],
    )()


# speedup vs baseline: 1.0014x; 1.0014x over previous
import functools
import math

import jax
import jax.numpy as jnp
from jax.experimental import pallas as pl
from jax.experimental.pallas import tpu as pltpu

_NUM_UNITS = 1024
_SCALE = float(_NUM_UNITS) ** 0.5
_NEG2LOG1E4_OVER_D = -2.0 * math.log(10000.0) / _NUM_UNITS
_HALF_PI = math.pi / 2.0
_SUB = 64
_STRIPE = 128


def _pe_body(out_ref, table_ref, s64_ref, c64_ref, sem_ref, *, seq: int, batch: int):
    ch1 = jax.lax.broadcasted_iota(jnp.int32, (1, _NUM_UNITS), 1)
    w1 = jnp.exp(ch1.astype(jnp.float32) * _NEG2LOG1E4_OVER_D)
    off = (ch1 % 2).astype(jnp.float32) * _HALF_PI

    r8 = jax.lax.broadcasted_iota(jnp.int32, (8, _NUM_UNITS), 0)
    ch8 = jax.lax.broadcasted_iota(jnp.int32, (8, _NUM_UNITS), 1)
    rw8 = r8.astype(jnp.float32) * jnp.exp(
        ch8.astype(jnp.float32) * _NEG2LOG1E4_OVER_D
    )
    s8 = jnp.sin(rw8)
    c8 = jnp.sin(rw8 + _HALF_PI)
    s64_ref[0:8, :] = s8
    c64_ref[0:8, :] = c8
    for q in range(1, _SUB // 8):
        ph = float(8 * q) * w1
        sq = jnp.sin(ph)
        cq = jnp.sin(ph + _HALF_PI)
        s64_ref[8 * q : 8 * q + 8, :] = sq * c8 + cq * s8
        c64_ref[8 * q : 8 * q + 8, :] = cq * c8 - sq * s8

    s64 = s64_ref[...]
    c64 = c64_ref[...]

    n_stripes = seq // _STRIPE

    def _copies(s):
        return [
            pltpu.make_async_copy(
                table_ref.at[pl.ds(s * _STRIPE, _STRIPE), :],
                out_ref.at[n, pl.ds(s * _STRIPE, _STRIPE), :],
                sem_ref.at[s % 4, n],
            )
            for n in range(batch)
        ]

    for s in range(n_stripes):
        for a in range(_STRIPE // _SUB):
            p0 = s * _STRIPE + a * _SUB
            phase = float(p0) * w1 + off
            sb = jnp.sin(phase) * _SCALE
            cb = jnp.sin(phase + _HALF_PI) * _SCALE
            row = pl.ds(p0, _SUB)
            table_ref[row, :] = sb * c64 + cb * s64
        if s == 0:
            table_ref[0:1, :] = jnp.zeros((1, _NUM_UNITS), jnp.float32)
        if s >= 4:
            for c in _copies(s - 4):
                c.wait()
        for c in _copies(s):
            c.start()
    for s in range(n_stripes - 4, n_stripes):
        for c in _copies(s):
            c.wait()


def kernel(inputs):
    n, t, d = inputs.shape
    body = functools.partial(_pe_body, seq=t, batch=n)
    return pl.pallas_call(
        body,
        out_shape=jax.ShapeDtypeStruct((n, t, d), jnp.float32),
        out_specs=pl.BlockSpec(memory_space=pl.ANY),
        scratch_shapes=[
            pltpu.VMEM((t, d), jnp.float32),
            pltpu.VMEM((_SUB, d), jnp.float32),
            pltpu.VMEM((_SUB, d), jnp.float32),
            pltpu.SemaphoreType.DMA((4, n)),
        ],
    )()
